# baseline (speedup 1.0000x reference)
import functools

import jax
import jax.numpy as jnp
from jax import lax
from jax.experimental import pallas as pl
from jax.experimental.pallas import tpu as pltpu
from jax.experimental.pallas import tpu_sc as plsc

_NF = 26
_V = 1000000
_B = 16384
_NW = 32
_BPW = _B // _NW
_L = 16
_CH = 128
_NCH = _BPW // _CH
_NC = _BPW // _L


def _tanh16(x):
    e = jnp.exp(2.0 * x)
    return (e - 1.0) / (e + 1.0)


_mesh = plsc.VectorSubcoreMesh(core_axis_name="c", subcore_axis_name="s")


@functools.partial(
    pl.kernel,
    mesh=_mesh,
    compiler_params=pltpu.CompilerParams(use_tc_tiling_on_sc=False),
    out_type=jax.ShapeDtypeStruct((_B,), jnp.float32),
    scratch_types=[
        pltpu.VMEM((_NF, _L), jnp.float32),
        pltpu.VMEM((_NF * _BPW,), jnp.int32),
        pltpu.VMEM((_NF * _BPW,), jnp.int32),
        pltpu.VMEM((_NF * _BPW,), jnp.int32),
        pltpu.VMEM((_BPW,), jnp.float32),
        pltpu.SemaphoreType.DMA,
        pltpu.SemaphoreType.DMA,
        pltpu.SemaphoreType.DMA,
    ],
)
def _sc_linear(xt_hbm, *rest):
    tabs = rest[:_NF]
    a_hbm = rest[_NF]
    out_hbm = rest[_NF + 1]
    wv, iv, iw, vv, acc, sem, g0, g1 = rest[_NF + 2:]
    cid = lax.axis_index("c")
    sid = lax.axis_index("s")
    wid = sid * 2 + cid
    base = wid * _BPW
    gsems = (g0, g1)

    stage = []
    for f in range(_NF):
        stage.append(
            pltpu.async_copy(xt_hbm.at[f, wid], iv.at[pl.ds(f * _BPW, _BPW)], sem)
        )
    pltpu.sync_copy(a_hbm, wv)
    for f in range(_NF):
        wv[f] = _tanh16(wv[f])

    def _build(f):
        def body(c, _):
            s = pl.ds(f * _BPW + c * _L, _L)
            iw[s] = iv[s] >> 1
            return 0

        lax.fori_loop(0, _NC, body, 0)

    def _fire(f):
        cps = []
        for j in range(_NCH):
            s = pl.ds(f * _BPW + j * _CH, _CH)
            cps.append(pltpu.async_copy(tabs[f].at[iw.at[s]], vv.at[s], gsems[f % 2]))
        return cps

    ln2 = jnp.broadcast_to(jnp.float32(0.6931471805599453), (_L,))
    c134 = jnp.broadcast_to(jnp.float32(134.0), (_L,))

    def _accum(f):
        wf = wv[f]
        first = f == 0

        def body(c, _):
            s = pl.ds(f * _BPW + c * _L, _L)
            sh = (iv[s] & 1) << 4
            bits = vv[s] >> sh
            e = (bits >> 7) & 0xFF
            m = bits & 0x7F
            sgn = 1 - ((bits >> 14) & 2)
            sm = (m + 128) * sgn
            val = sm.astype(jnp.float32) * jnp.exp(
                (e.astype(jnp.float32) - c134) * ln2
            )
            d = pl.ds(c * _L, _L)
            if first:
                acc[d] = val * wf
            else:
                acc[d] = acc[d] + val * wf
            return 0

        lax.fori_loop(0, _NC, body, 0)

    gcps = None
    for f in range(_NF):
        stage[f].wait()
        _build(f)
        nxt = _fire(f)
        if gcps is not None:
            for cp in gcps:
                cp.wait()
            _accum(f - 1)
        gcps = nxt
    for cp in gcps:
        cp.wait()
    _accum(_NF - 1)

    pltpu.sync_copy(acc, out_hbm.at[pl.ds(base, _BPW)])


def kernel(X, tables, alpha):
    xt = X.T.reshape(_NF, _NW, _BPW)
    a_b = jnp.broadcast_to(alpha[:, None], (_NF, _L))
    tabs = [
        lax.bitcast_convert_type(
            tables[f, :, 0].astype(jnp.bfloat16).reshape(_V // 2, 2), jnp.int32
        )
        for f in range(_NF)
    ]
    out = _sc_linear(xt, *tabs, a_b)
    return out.reshape(_B, 1)


# speedup vs baseline: 20.6604x; 20.6604x over previous
import functools

import jax
import jax.numpy as jnp
from jax import lax
from jax.experimental import pallas as pl
from jax.experimental.pallas import tpu as pltpu
from jax.experimental.pallas import tpu_sc as plsc

_NF = 26
_V = 1000000
_B = 16384
_NW = 32
_BPW = _B // _NW
_L = 16
_CH = 128
_NCH = _BPW // _CH
_NC = _BPW // _L


def _tanh16(x):
    e = jnp.exp(2.0 * x)
    return (e - 1.0) / (e + 1.0)


_mesh = plsc.VectorSubcoreMesh(core_axis_name="c", subcore_axis_name="s")


@functools.partial(
    pl.kernel,
    mesh=_mesh,
    compiler_params=pltpu.CompilerParams(use_tc_tiling_on_sc=False),
    out_type=jax.ShapeDtypeStruct((_B,), jnp.float32),
    scratch_types=[
        pltpu.VMEM((_NF, _L), jnp.float32),
        pltpu.VMEM((_NF * _BPW,), jnp.int32),
        pltpu.VMEM((_NF * _BPW,), jnp.float32),
        pltpu.VMEM((_BPW,), jnp.float32),
        pltpu.SemaphoreType.DMA,
        pltpu.SemaphoreType.DMA,
        pltpu.SemaphoreType.DMA,
    ],
)
def _sc_linear(xt_hbm, *rest):
    tabs = rest[:_NF]
    a_hbm = rest[_NF]
    out_hbm = rest[_NF + 1]
    wv, iv, vv, acc, sem, g0, g1 = rest[_NF + 2:]
    cid = lax.axis_index("c")
    sid = lax.axis_index("s")
    wid = sid * 2 + cid
    base = wid * _BPW
    gsems = (g0, g1)

    stage = []
    for f in range(_NF):
        stage.append(
            pltpu.async_copy(xt_hbm.at[f, wid], iv.at[pl.ds(f * _BPW, _BPW)], sem)
        )
    pltpu.sync_copy(a_hbm, wv)
    for f in range(_NF):
        wv[f] = _tanh16(wv[f])

    def _fire(f):
        cps = []
        for j in range(_NCH):
            s = pl.ds(f * _BPW + j * _CH, _CH)
            cps.append(pltpu.async_copy(tabs[f].at[iv.at[s]], vv.at[s], gsems[f % 2]))
        return cps

    def _accum(f):
        wf = wv[f]
        if f == 0:
            def body0(c, _):
                s = pl.ds(c * _L, _L)
                acc[s] = vv[s] * wf
                return 0

            lax.fori_loop(0, _NC, body0, 0)
        else:
            def body(c, _):
                s = pl.ds(c * _L, _L)
                acc[s] = acc[s] + vv[pl.ds(f * _BPW + c * _L, _L)] * wf
                return 0

            lax.fori_loop(0, _NC, body, 0)

    gcps = None
    for f in range(_NF):
        stage[f].wait()
        nxt = _fire(f)
        if gcps is not None:
            for cp in gcps:
                cp.wait()
            _accum(f - 1)
        gcps = nxt
    for cp in gcps:
        cp.wait()
    _accum(_NF - 1)

    pltpu.sync_copy(acc, out_hbm.at[pl.ds(base, _BPW)])


def kernel(X, tables, alpha):
    xt = X.T.reshape(_NF, _NW, _BPW)
    a_b = jnp.broadcast_to(alpha[:, None], (_NF, _L))
    tabs = [tables[f, :, 0] for f in range(_NF)]
    out = _sc_linear(xt, *tabs, a_b)
    return out.reshape(_B, 1)
